# baseline (speedup 1.0000x reference)
import jax
import jax.numpy as jnp
from jax.experimental import pallas as pl
from jax.experimental.pallas import tpu as pltpu


def _slab_kernel(x_ref, out_ref):
    jj = jax.lax.broadcasted_iota(jnp.int32, (128, 128), 0)
    kk = jax.lax.broadcasted_iota(jnp.int32, (128, 128), 1)
    interior = (jj < 127) & (kk < 127)

    def slice_term(a, xn):
        dx = xn - a
        ay = pltpu.roll(a, 127, axis=0)
        az = pltpu.roll(a, 127, axis=1)
        dy = ay - a
        dz = az - a
        return dx * dx + dy * dy + dz * dz

    def body(m, carry):
        acc0, acc1, a = carry
        base = m * 8
        vals = [a] + [x_ref[0, base + t] for t in range(1, 9)]
        for t in range(0, 8, 2):
            acc0 = acc0 + slice_term(vals[t], vals[t + 1])
            acc1 = acc1 + slice_term(vals[t + 1], vals[t + 2])
        return acc0, acc1, vals[8]

    zero = jnp.zeros((128, 128), dtype=jnp.float32)
    acc0, acc1, a = jax.lax.fori_loop(0, 15, body, (zero, zero, x_ref[0, 0]))
    for i in range(120, 127):
        xn = x_ref[0, i + 1]
        if i % 2 == 0:
            acc0 = acc0 + slice_term(a, xn)
        else:
            acc1 = acc1 + slice_term(a, xn)
        a = xn
    acc = acc0 + acc1
    acc = jnp.where(interior, acc, 0.0)
    r = jnp.sum(acc, axis=0, keepdims=True)
    out_ref[0] = jnp.sum(r, axis=1, keepdims=True)


def kernel(mesh):
    slabs = mesh.reshape(12, 128, 128, 128)
    partials = pl.pallas_call(
        _slab_kernel,
        grid=(12,),
        in_specs=[
            pl.BlockSpec((1, 128, 128, 128), lambda i: (i, 0, 0, 0)),
        ],
        out_specs=pl.BlockSpec((1, 1, 1), lambda i: (i, 0, 0)),
        out_shape=jax.ShapeDtypeStruct((12, 1, 1), jnp.float32),
        compiler_params=pltpu.CompilerParams(
            dimension_semantics=("parallel",),
        ),
    )(slabs)
    return jnp.sum(partials) / jnp.float32(12.0)


# speedup vs baseline: 1.0799x; 1.0799x over previous
import jax
import jax.numpy as jnp
from jax.experimental import pallas as pl
from jax.experimental.pallas import tpu as pltpu


def _slab_kernel(x_ref, out_ref):
    jj = jax.lax.broadcasted_iota(jnp.int32, (128, 128), 0)
    kk = jax.lax.broadcasted_iota(jnp.int32, (128, 128), 1)
    interior = (jj < 127) & (kk < 127)

    accs = [jnp.zeros((128, 128), dtype=jnp.float32) for _ in range(2)]
    a = x_ref[0, 0]
    for i in range(127):
        xn = x_ref[0, i + 1]
        dx = xn - a
        ay = pltpu.roll(a, 127, axis=0)
        az = pltpu.roll(a, 127, axis=1)
        dy = ay - a
        dz = az - a
        s = dx * dx + dy * dy + dz * dz
        accs[i % 2] = accs[i % 2] + s
        a = xn
    acc = accs[0] + accs[1]
    acc = jnp.where(interior, acc, 0.0)
    r = jnp.sum(acc, axis=0, keepdims=True)
    out_ref[0] = jnp.sum(r, axis=1, keepdims=True)


def kernel(mesh):
    slabs = mesh.reshape(12, 128, 128, 128)
    partials = pl.pallas_call(
        _slab_kernel,
        grid=(12,),
        in_specs=[
            pl.BlockSpec((1, 128, 128, 128), lambda i: (i, 0, 0, 0)),
        ],
        out_specs=pl.BlockSpec((1, 1, 1), lambda i: (i, 0, 0)),
        out_shape=jax.ShapeDtypeStruct((12, 1, 1), jnp.float32),
        compiler_params=pltpu.CompilerParams(
            dimension_semantics=("parallel",),
        ),
    )(slabs)
    return jnp.sum(partials) / jnp.float32(12.0)
